# baseline (speedup 1.0000x reference)
import functools

import numpy as np
import jax
import jax.numpy as jnp
from jax import lax
from jax.experimental import pallas as pl
from jax.experimental.pallas import tpu as pltpu
from jax.experimental.pallas import tpu_sc as plsc

EMBED = 256
HEADS = 8
LEVELS = 4
POINTS = 4
HD = EMBED // HEADS
LP = LEVELS * POINTS
SPATIAL = [(64, 64), (32, 32), (16, 16), (8, 8)]
NV = sum(h * w for h, w in SPATIAL)
BS = 2
NQ = NV
TOT_ROWS = BS * NV * HEADS

BQ = 680
NQB = NQ // BQ

NW = 32
ITEMS = BS * NQ * HEADS
IW = ITEMS // NW
CI = 16
CH = IW // CI
ROWS_PER_CHUNK = CI * 4 * LP
IDX_ROWS = ROWS_PER_CHUNK // 128

_lvl = np.tile(np.repeat(np.arange(LEVELS), POINTS), HEADS)
_hh = np.repeat(np.arange(HEADS), LP)
_Wnp = np.array([w for (h, w) in SPATIAL], np.float32)[_lvl]
_Hnp = np.array([h for (h, w) in SPATIAL], np.float32)[_lvl]
_off = np.cumsum([0] + [h * w for h, w in SPATIAL])[:LEVELS]
_base = _off[_lvl].astype(np.int32)
_hlane = _hh.astype(np.int32)
_BD = np.kron(np.eye(HEADS, dtype=np.float32),
              np.ones((LP, LP), np.float32))

_cp = np.arange(HD)
_cnat = np.where(_cp < 16, 2 * _cp, 2 * (_cp - 16) + 1)
_operm = (np.repeat(np.arange(HEADS), HD) * HD + np.tile(_cnat, HEADS)).astype(np.int32)


def _vproj_body(x_ref, w_ref, b_ref, out_ref):
    out_ref[0] = (
        jnp.dot(x_ref[0], w_ref[...], preferred_element_type=jnp.float32)
        + b_ref[0]).astype(jnp.bfloat16)


def _prep_body(q_ref, wall_ref, ball_ref, refx_ref, refy_ref,
               wl_ref, hl_ref, base_ref, hlane_ref, bd_ref,
               idx_out, w_out):
    b = pl.program_id(0)
    q = q_ref[0]
    t = jnp.dot(q, wall_ref[...], preferred_element_type=jnp.float32) + ball_ref[0]
    sox = t[:, 0:128]
    soy = t[:, 128:256]
    awr = t[:, 256:384]
    e = jnp.exp(awr)
    s = jnp.dot(e, bd_ref[...], preferred_element_type=jnp.float32)
    aw = e / s

    wv = wl_ref[0]
    hv = hl_ref[0]
    px = refx_ref[0] * wv + sox - 0.5
    py = refy_ref[0] * hv + soy - 0.5
    x0 = jnp.floor(px)
    y0 = jnp.floor(py)
    wx1 = px - x0
    wx0 = 1.0 - wx1
    wy1 = py - y0
    wy0 = 1.0 - wy1
    vx0 = ((x0 >= 0.0) & (x0 < wv)).astype(jnp.float32)
    vx1 = ((x0 >= -1.0) & (x0 < wv - 1.0)).astype(jnp.float32)
    vy0 = ((y0 >= 0.0) & (y0 < hv)).astype(jnp.float32)
    vy1 = ((y0 >= -1.0) & (y0 < hv - 1.0)).astype(jnp.float32)
    wi = wv.astype(jnp.int32)
    xi0 = jnp.clip(x0, 0.0, wv - 1.0).astype(jnp.int32)
    xi1 = jnp.clip(x0 + 1.0, 0.0, wv - 1.0).astype(jnp.int32)
    yi0 = jnp.clip(y0, 0.0, hv - 1.0).astype(jnp.int32)
    yi1 = jnp.clip(y0 + 1.0, 0.0, hv - 1.0).astype(jnp.int32)
    bbase = b * NV
    r0 = base_ref[0] + yi0 * wi + bbase
    r1 = base_ref[0] + yi1 * wi + bbase
    hl8 = hlane_ref[0]
    idx_out[0, :, 0] = jnp.clip((r0 + xi0) * HEADS + hl8, 0, TOT_ROWS - 1)
    idx_out[0, :, 1] = jnp.clip((r0 + xi1) * HEADS + hl8, 0, TOT_ROWS - 1)
    idx_out[0, :, 2] = jnp.clip((r1 + xi0) * HEADS + hl8, 0, TOT_ROWS - 1)
    idx_out[0, :, 3] = jnp.clip((r1 + xi1) * HEADS + hl8, 0, TOT_ROWS - 1)
    w_out[0, :, 0] = aw * (wx0 * wy0) * (vx0 * vy0)
    w_out[0, :, 1] = aw * (wx1 * wy0) * (vx1 * vy0)
    w_out[0, :, 2] = aw * (wx0 * wy1) * (vx0 * vy1)
    w_out[0, :, 3] = aw * (wx1 * wy1) * (vx1 * vy1)


def _post_body(s_ref, w_ref, b_ref, q_ref, out_ref):
    out_ref[0] = (
        jnp.dot(s_ref[0], w_ref[...], preferred_element_type=jnp.float32)
        + b_ref[0] + q_ref[0])


def _sc_body(table, idxh, wh, out, idx_v, w_v, rows_v, out_v, semg0, semg1, sem_o):
    wid = lax.axis_index("s") * 2 + lax.axis_index("c")
    semg = [semg0, semg1]

    def load_and_fire(j, par):
        pltpu.sync_copy(idxh.at[wid, j], idx_v.at[par])
        pltpu.sync_copy(wh.at[wid, j], w_v.at[par])
        for d in range(IDX_ROWS):
            pltpu.async_copy(table.at[idx_v.at[par, d]],
                             rows_v.at[par, pl.ds(d * 128, 128)], semg[par])

    def drain(par):
        for d in range(IDX_ROWS):
            pltpu.make_async_copy(
                table.at[idx_v.at[par, d]],
                rows_v.at[par, pl.ds(d * 128, 128)], semg[par]).wait()

    load_and_fire(0, 0)

    def outer(j2, carry):
        for par in range(2):
            j = j2 * 2 + par

            @pl.when(j < CH - 1)
            def _():
                load_and_fire(j + 1, 1 - par)

            drain(par)

            def item(t, c2):
                qq = t >> 3
                h = t & 7
                acc0 = jnp.zeros((16,), jnp.float32)
                acc1 = jnp.zeros((16,), jnp.float32)
                for g in range(4):
                    row8 = qq * 4 + g
                    col = h * 16
                    wv16 = w_v[par, row8, pl.ds(col, 16)]
                    rbase = row8 * 128 + col
                    for k in range(16):
                        wk = wv16[k]
                        u = rows_v[par, rbase + k, :]
                        fe = plsc.bitcast(u << 16, jnp.float32)
                        fo = plsc.bitcast(u & jnp.uint32(0xFFFF0000), jnp.float32)
                        acc0 = acc0 + wk * fe
                        acc1 = acc1 + wk * fo
                out_v[t, pl.ds(0, 16)] = acc0
                out_v[t, pl.ds(16, 16)] = acc1
                return c2

            lax.fori_loop(0, CI, item, 0)
            cp = pltpu.async_copy(out_v, out.at[wid, j], sem_o)
            cp.wait()
        return carry

    lax.fori_loop(0, CH // 2, outer, 0)


def _build_table(value, W_vp, b_vp):
    vproj = pl.pallas_call(
        _vproj_body,
        grid=(BS, NQB),
        in_specs=[
            pl.BlockSpec((1, BQ, EMBED), lambda b, i: (b, i, 0)),
            pl.BlockSpec((EMBED, EMBED), lambda b, i: (0, 0)),
            pl.BlockSpec((1, EMBED), lambda b, i: (0, 0)),
        ],
        out_specs=pl.BlockSpec((1, BQ, EMBED), lambda b, i: (b, i, 0)),
        out_shape=jax.ShapeDtypeStruct((BS, NV, EMBED), jnp.bfloat16),
    )(value, W_vp.T, b_vp.reshape(1, EMBED))
    return lax.bitcast_convert_type(
        vproj.reshape(TOT_ROWS, HD // 2, 2), jnp.uint32)


def _build_idx_w(query, reference_points, W_so, b_so, W_aw, b_aw):
    wall_t = jnp.concatenate([W_so[0::2], W_so[1::2], W_aw], axis=0).T
    ball = jnp.concatenate([b_so[0::2], b_so[1::2], b_aw]).reshape(1, 384)
    refx = jnp.tile(jnp.repeat(reference_points[..., 0], POINTS, axis=-1),
                    (1, 1, HEADS))
    refy = jnp.tile(jnp.repeat(reference_points[..., 1], POINTS, axis=-1),
                    (1, 1, HEADS))
    wl = jnp.asarray(_Wnp).reshape(1, 128)
    hl = jnp.asarray(_Hnp).reshape(1, 128)
    basev = jnp.asarray(_base).reshape(1, 128)
    hlane = jnp.asarray(_hlane).reshape(1, 128)
    bd = jnp.asarray(_BD)

    qspec = pl.BlockSpec((1, BQ, EMBED), lambda b, i: (b, i, 0))
    cspec128 = pl.BlockSpec((1, 128), lambda b, i: (0, 0))
    lspec = pl.BlockSpec((1, BQ, 128), lambda b, i: (b, i, 0))
    ospec = pl.BlockSpec((1, BQ, 4, 128), lambda b, i: (b, i, 0, 0))
    idx4, w4 = pl.pallas_call(
        _prep_body,
        grid=(BS, NQB),
        in_specs=[
            qspec,
            pl.BlockSpec((EMBED, 384), lambda b, i: (0, 0)),
            pl.BlockSpec((1, 384), lambda b, i: (0, 0)),
            lspec, lspec,
            cspec128, cspec128, cspec128, cspec128,
            pl.BlockSpec((128, 128), lambda b, i: (0, 0)),
        ],
        out_specs=[ospec, ospec],
        out_shape=[
            jax.ShapeDtypeStruct((BS, NQ, 4, 128), jnp.int32),
            jax.ShapeDtypeStruct((BS, NQ, 4, 128), jnp.float32),
        ],
    )(query, wall_t, ball, refx, refy, wl, hl, basev, hlane, bd)
    idx_sc = idx4.reshape(NW, CH, IDX_ROWS, 128)
    w_sc = w4.reshape(NW, CH, IDX_ROWS, 128)
    return idx_sc, w_sc


def _sc_gather(table, idx_sc, w_sc):
    mesh = plsc.VectorSubcoreMesh(core_axis_name="c", subcore_axis_name="s")
    f = functools.partial(
        pl.kernel,
        mesh=mesh,
        compiler_params=pltpu.CompilerParams(use_tc_tiling_on_sc=False,
                                             needs_layout_passes=False),
        out_type=jax.ShapeDtypeStruct((NW, CH, CI, HD), jnp.float32),
        scratch_types=[
            pltpu.VMEM((2, IDX_ROWS, 128), jnp.int32),
            pltpu.VMEM((2, IDX_ROWS, 128), jnp.float32),
            pltpu.VMEM((2, ROWS_PER_CHUNK, HD // 2), jnp.uint32),
            pltpu.VMEM((CI, HD), jnp.float32),
            pltpu.SemaphoreType.DMA,
            pltpu.SemaphoreType.DMA,
            pltpu.SemaphoreType.DMA,
        ],
    )(_sc_body)
    return f(table, idx_sc, w_sc)


def _post(sc_out, W_op, b_op, query):
    return pl.pallas_call(
        _post_body,
        grid=(BS, NQB),
        in_specs=[
            pl.BlockSpec((1, BQ, EMBED), lambda b, i: (b, i, 0)),
            pl.BlockSpec((EMBED, EMBED), lambda b, i: (0, 0)),
            pl.BlockSpec((1, EMBED), lambda b, i: (0, 0)),
            pl.BlockSpec((1, BQ, EMBED), lambda b, i: (b, i, 0)),
        ],
        out_specs=pl.BlockSpec((1, BQ, EMBED), lambda b, i: (b, i, 0)),
        out_shape=jax.ShapeDtypeStruct((BS, NQ, EMBED), jnp.float32),
    )(sc_out, W_op.T[jnp.asarray(_operm)], b_op.reshape(1, EMBED), query)


def kernel(query, value, reference_points, spatial_shapes, level_start_index,
           W_so, b_so, W_aw, b_aw, W_vp, b_vp, W_op, b_op):
    table = _build_table(value, W_vp, b_vp)
    idx_sc, w_sc = _build_idx_w(query, reference_points, W_so, b_so, W_aw, b_aw)
    sc_out = _sc_gather(table, idx_sc, w_sc)
    sc_out = sc_out.reshape(BS, NQ, EMBED)
    return _post(sc_out, W_op, b_op, query)


# speedup vs baseline: 5.4200x; 5.4200x over previous
import functools

import numpy as np
import jax
import jax.numpy as jnp
from jax import lax
from jax.experimental import pallas as pl
from jax.experimental.pallas import tpu as pltpu
from jax.experimental.pallas import tpu_sc as plsc

EMBED = 256
HEADS = 8
LEVELS = 4
POINTS = 4
HD = EMBED // HEADS
LP = LEVELS * POINTS
SPATIAL = [(64, 64), (32, 32), (16, 16), (8, 8)]
NV = sum(h * w for h, w in SPATIAL)
BS = 2
NQ = NV
TAB_ROWS = NV * HEADS

BQ = 680
NQB = NQ // BQ

NW = 32
ITEMS = NQ * HEADS
IW = ITEMS // NW
CI = 16
CH = IW // CI
ROWS_PER_CHUNK = CI * 4 * LP
IDX_ROWS = ROWS_PER_CHUNK // 128

_lvl = np.tile(np.repeat(np.arange(LEVELS), POINTS), HEADS)
_hh = np.repeat(np.arange(HEADS), LP)
_Wnp = np.array([w for (h, w) in SPATIAL], np.float32)[_lvl]
_Hnp = np.array([h for (h, w) in SPATIAL], np.float32)[_lvl]
_off = np.cumsum([0] + [h * w for h, w in SPATIAL])[:LEVELS]
_base = _off[_lvl].astype(np.int32)
_hlane = _hh.astype(np.int32)
_BD = np.kron(np.eye(HEADS, dtype=np.float32),
              np.ones((LP, LP), np.float32))


def _fused_body(x_ref, q_ref, wvp_ref, bvp_ref, wall_ref, ball_ref,
                refx_ref, refy_ref, wl_ref, hl_ref, base_ref, hlane_ref,
                bd_ref, v_out, iw_out):
    y = (jnp.dot(x_ref[0], wvp_ref[...], preferred_element_type=jnp.float32)
         + bvp_ref[0])
    u = lax.bitcast_convert_type(y, jnp.uint32)
    r = (u + jnp.uint32(0x7FFF) + ((u >> 16) & jnp.uint32(1))) >> 16
    packed = [r[:, h * HD:h * HD + 16] | (r[:, h * HD + 16:(h + 1) * HD] << 16)
              for h in range(HEADS)]
    v_out[0] = jnp.concatenate(packed, axis=1)

    q = q_ref[0]
    t = jnp.dot(q, wall_ref[...], preferred_element_type=jnp.float32) + ball_ref[0]
    sox = t[:, 0:128]
    soy = t[:, 128:256]
    awr = t[:, 256:384]
    e = jnp.exp(awr)
    s = jnp.dot(e, bd_ref[...], preferred_element_type=jnp.float32)
    aw = e / s

    wv = wl_ref[0]
    hv = hl_ref[0]
    px = refx_ref[0] * wv + sox - 0.5
    py = refy_ref[0] * hv + soy - 0.5
    x0 = jnp.floor(px)
    y0 = jnp.floor(py)
    wx1 = px - x0
    wx0 = 1.0 - wx1
    wy1 = py - y0
    wy0 = 1.0 - wy1
    vx0 = ((x0 >= 0.0) & (x0 < wv)).astype(jnp.float32)
    vx1 = ((x0 >= -1.0) & (x0 < wv - 1.0)).astype(jnp.float32)
    vy0 = ((y0 >= 0.0) & (y0 < hv)).astype(jnp.float32)
    vy1 = ((y0 >= -1.0) & (y0 < hv - 1.0)).astype(jnp.float32)
    wi = wv.astype(jnp.int32)
    xi0 = jnp.clip(x0, 0.0, wv - 1.0).astype(jnp.int32)
    xi1 = jnp.clip(x0 + 1.0, 0.0, wv - 1.0).astype(jnp.int32)
    yi0 = jnp.clip(y0, 0.0, hv - 1.0).astype(jnp.int32)
    yi1 = jnp.clip(y0 + 1.0, 0.0, hv - 1.0).astype(jnp.int32)
    r0 = base_ref[0] + yi0 * wi
    r1 = base_ref[0] + yi1 * wi
    hl8 = hlane_ref[0]
    iw_out[0, :, 0] = jnp.clip((r0 + xi0) * HEADS + hl8, 0, TAB_ROWS - 1)
    iw_out[0, :, 1] = jnp.clip((r0 + xi1) * HEADS + hl8, 0, TAB_ROWS - 1)
    iw_out[0, :, 2] = jnp.clip((r1 + xi0) * HEADS + hl8, 0, TAB_ROWS - 1)
    iw_out[0, :, 3] = jnp.clip((r1 + xi1) * HEADS + hl8, 0, TAB_ROWS - 1)
    bc = lambda x: lax.bitcast_convert_type(x, jnp.int32)
    iw_out[0, :, 4] = bc(aw * (wx0 * wy0) * (vx0 * vy0))
    iw_out[0, :, 5] = bc(aw * (wx1 * wy0) * (vx1 * vy0))
    iw_out[0, :, 6] = bc(aw * (wx0 * wy1) * (vx0 * vy1))
    iw_out[0, :, 7] = bc(aw * (wx1 * wy1) * (vx1 * vy1))


def _post_body(s_ref, w_ref, b_ref, q_ref, out_ref):
    out_ref[0] = (
        jnp.dot(s_ref[0], w_ref[...], preferred_element_type=jnp.float32)
        + b_ref[0] + q_ref[0])


def _sc_body(table, iwh, out, iw_v, rows_v, out_v, semg0, semg1, sem_o, sem_iw):
    wid = lax.axis_index("s") * 2 + lax.axis_index("c")
    semg = [semg0, semg1]
    srcrows = [qq * 8 + c4 for qq in range(2) for c4 in range(4)]

    def fire_iw(j, par):
        pltpu.async_copy(iwh.at[wid, j], iw_v.at[par], sem_iw)

    def wait_iw(par):
        pltpu.make_async_copy(iwh.at[wid, 0], iw_v.at[par], sem_iw).wait()

    def fire_gathers(par):
        for d in range(IDX_ROWS):
            pltpu.async_copy(table.at[iw_v.at[par, srcrows[d]]],
                             rows_v.at[par, pl.ds(d * 128, 128)], semg[par])

    def drain_gathers(par):
        for d in range(IDX_ROWS):
            pltpu.make_async_copy(
                table.at[iw_v.at[par, srcrows[d]]],
                rows_v.at[par, pl.ds(d * 128, 128)], semg[par]).wait()

    def wait_out(j, par):
        pltpu.make_async_copy(out_v.at[par], out.at[wid, j], sem_o).wait()

    def chunk_step(j, par):
        @pl.when(j < CH - 1)
        def _():
            wait_iw(1 - par)
            fire_gathers(1 - par)

        drain_gathers(par)

        def item(t, c2):
            qq = t >> 3
            h = t & 7
            a0 = jnp.zeros((16,), jnp.float32)
            b0 = jnp.zeros((16,), jnp.float32)
            a1 = jnp.zeros((16,), jnp.float32)
            b1 = jnp.zeros((16,), jnp.float32)
            for g in range(4):
                col = h * 16
                wv16 = plsc.bitcast(
                    iw_v[par, qq * 8 + 4 + g, pl.ds(col, 16)], jnp.float32)
                rbase = (qq * 4 + g) * 128 + col
                for k in range(16):
                    wk = wv16[k]
                    u = rows_v[par, rbase + k, :]
                    fe = plsc.bitcast(u << 16, jnp.float32)
                    fo = plsc.bitcast(u, jnp.float32)
                    if k & 1:
                        a1 = a1 + wk * fe
                        b1 = b1 + wk * fo
                    else:
                        a0 = a0 + wk * fe
                        b0 = b0 + wk * fo
            out_v[par, t, pl.ds(0, 16)] = a0 + a1
            out_v[par, t, pl.ds(16, 16)] = b0 + b1
            return c2

        lax.fori_loop(0, CI, item, 0)

        @pl.when(j + 2 < CH)
        def _():
            fire_iw(j + 2, par)

        @pl.when(j >= 2)
        def _():
            wait_out(j - 2, par)

        pltpu.async_copy(out_v.at[par], out.at[wid, j], sem_o)

    fire_iw(0, 0)
    wait_iw(0)
    fire_gathers(0)
    fire_iw(1, 1)

    chunk_step(0, 0)

    def outer(j2, carry):
        j = 1 + j2 * 2
        chunk_step(j, 1)
        chunk_step(j + 1, 0)
        return carry

    lax.fori_loop(0, (CH - 1) // 2, outer, 0)
    wait_out(CH - 2, 1)
    wait_out(CH - 1, 0)


def _fused_prep(b, value, query, reference_points, W_vp, b_vp, wall_t, ball):
    refx = jnp.tile(jnp.repeat(reference_points[b, :, :, 0], POINTS, axis=-1),
                    (1, HEADS)).reshape(1, NQ, 128)
    refy = jnp.tile(jnp.repeat(reference_points[b, :, :, 1], POINTS, axis=-1),
                    (1, HEADS)).reshape(1, NQ, 128)
    wl = jnp.asarray(_Wnp).reshape(1, 128)
    hl = jnp.asarray(_Hnp).reshape(1, 128)
    basev = jnp.asarray(_base).reshape(1, 128)
    hlane = jnp.asarray(_hlane).reshape(1, 128)
    bd = jnp.asarray(_BD)

    bspec = pl.BlockSpec((1, BQ, EMBED), lambda i: (0, i, 0))
    cspec128 = pl.BlockSpec((1, 128), lambda i: (0, 0))
    lspec = pl.BlockSpec((1, BQ, 128), lambda i: (0, i, 0))
    vtab, iw = pl.pallas_call(
        _fused_body,
        grid=(NQB,),
        in_specs=[
            bspec, bspec,
            pl.BlockSpec((EMBED, EMBED), lambda i: (0, 0)),
            pl.BlockSpec((1, EMBED), lambda i: (0, 0)),
            pl.BlockSpec((EMBED, 384), lambda i: (0, 0)),
            pl.BlockSpec((1, 384), lambda i: (0, 0)),
            lspec, lspec,
            cspec128, cspec128, cspec128, cspec128,
            pl.BlockSpec((128, 128), lambda i: (0, 0)),
        ],
        out_specs=[
            pl.BlockSpec((1, BQ, EMBED // 2), lambda i: (0, i, 0)),
            pl.BlockSpec((1, BQ, 8, 128), lambda i: (0, i, 0, 0)),
        ],
        out_shape=[
            jax.ShapeDtypeStruct((1, NV, EMBED // 2), jnp.uint32),
            jax.ShapeDtypeStruct((1, NQ, 8, 128), jnp.int32),
        ],
    )(value[b][None], query[b][None], W_vp.T, b_vp.reshape(1, EMBED),
      wall_t, ball, refx, refy, wl, hl, basev, hlane, bd)
    return vtab.reshape(TAB_ROWS, HD // 2), iw.reshape(NW, CH, 16, 128)


def _sc_gather(table, iw_sc):
    mesh = plsc.VectorSubcoreMesh(core_axis_name="c", subcore_axis_name="s")
    f = functools.partial(
        pl.kernel,
        mesh=mesh,
        compiler_params=pltpu.CompilerParams(use_tc_tiling_on_sc=False,
                                             needs_layout_passes=False),
        out_type=jax.ShapeDtypeStruct((NW, CH, CI, HD), jnp.float32),
        scratch_types=[
            pltpu.VMEM((2, 16, 128), jnp.int32),
            pltpu.VMEM((2, ROWS_PER_CHUNK, HD // 2), jnp.uint32),
            pltpu.VMEM((2, CI, HD), jnp.float32),
            pltpu.SemaphoreType.DMA,
            pltpu.SemaphoreType.DMA,
            pltpu.SemaphoreType.DMA,
            pltpu.SemaphoreType.DMA,
        ],
    )(_sc_body)
    return f(table, iw_sc)


def _post(b, sc_out, W_opt_perm, b_op, query):
    return pl.pallas_call(
        _post_body,
        grid=(NQB,),
        in_specs=[
            pl.BlockSpec((1, BQ, EMBED), lambda i: (0, i, 0)),
            pl.BlockSpec((EMBED, EMBED), lambda i: (0, 0)),
            pl.BlockSpec((1, EMBED), lambda i: (0, 0)),
            pl.BlockSpec((1, BQ, EMBED), lambda i: (0, i, 0)),
        ],
        out_specs=pl.BlockSpec((1, BQ, EMBED), lambda i: (0, i, 0)),
        out_shape=jax.ShapeDtypeStruct((1, NQ, EMBED), jnp.float32),
    )(sc_out.reshape(1, NQ, EMBED), W_opt_perm, b_op.reshape(1, EMBED),
      query[b][None])


def kernel(query, value, reference_points, spatial_shapes, level_start_index,
           W_so, b_so, W_aw, b_aw, W_vp, b_vp, W_op, b_op):
    wall_t = jnp.concatenate([W_so[0::2], W_so[1::2], W_aw], axis=0).T
    ball = jnp.concatenate([b_so[0::2], b_so[1::2], b_aw]).reshape(1, 384)
    outs = []
    for b in range(BS):
        table, iw_sc = _fused_prep(b, value, query, reference_points,
                                   W_vp, b_vp, wall_t, ball)
        sc_out = _sc_gather(table, iw_sc)
        outs.append(_post(b, sc_out, W_op.T, b_op, query))
    return jnp.concatenate(outs, axis=0)


# speedup vs baseline: 5.6124x; 1.0355x over previous
import functools

import numpy as np
import jax
import jax.numpy as jnp
from jax import lax
from jax.experimental import pallas as pl
from jax.experimental.pallas import tpu as pltpu
from jax.experimental.pallas import tpu_sc as plsc

EMBED = 256
HEADS = 8
LEVELS = 4
POINTS = 4
HD = EMBED // HEADS
LP = LEVELS * POINTS
SPATIAL = [(64, 64), (32, 32), (16, 16), (8, 8)]
NV = sum(h * w for h, w in SPATIAL)
BS = 2
NQ = NV
TOT_ROWS = BS * NV * HEADS

BQ = 680
NQB = NQ // BQ

NW = 32
ITEMS = BS * NQ * HEADS
IW = ITEMS // NW
CI = 16
CH = IW // CI
ROWS_PER_CHUNK = CI * 4 * LP
IDX_ROWS = ROWS_PER_CHUNK // 128

_lvl = np.tile(np.repeat(np.arange(LEVELS), POINTS), HEADS)
_hh = np.repeat(np.arange(HEADS), LP)
_Wnp = np.array([w for (h, w) in SPATIAL], np.float32)[_lvl]
_Hnp = np.array([h for (h, w) in SPATIAL], np.float32)[_lvl]
_off = np.cumsum([0] + [h * w for h, w in SPATIAL])[:LEVELS]
_base = _off[_lvl].astype(np.int32)
_hlane = _hh.astype(np.int32)
_BD = np.kron(np.eye(HEADS, dtype=np.float32),
              np.ones((LP, LP), np.float32))


def _fused_body(x_ref, q_ref, wvp_ref, bvp_ref, wall_ref, ball_ref,
                refx_ref, refy_ref, wl_ref, hl_ref, base_ref, hlane_ref,
                bd_ref, v_out, iw_out):
    y = (jnp.dot(x_ref[0], wvp_ref[...], preferred_element_type=jnp.float32)
         + bvp_ref[0])
    u = lax.bitcast_convert_type(y, jnp.uint32)
    r = (u + jnp.uint32(0x7FFF) + ((u >> 16) & jnp.uint32(1))) >> 16
    packed = [r[:, h * HD:h * HD + 16] | (r[:, h * HD + 16:(h + 1) * HD] << 16)
              for h in range(HEADS)]
    v_out[0] = jnp.concatenate(packed, axis=1)

    b = pl.program_id(0)
    q = q_ref[0]
    t = jnp.dot(q, wall_ref[...], preferred_element_type=jnp.float32) + ball_ref[0]
    sox = t[:, 0:128]
    soy = t[:, 128:256]
    awr = t[:, 256:384]
    e = jnp.exp(awr)
    s = jnp.dot(e, bd_ref[...], preferred_element_type=jnp.float32)
    aw = e / s

    wv = wl_ref[0]
    hv = hl_ref[0]
    px = refx_ref[0] * wv + sox - 0.5
    py = refy_ref[0] * hv + soy - 0.5
    x0 = jnp.floor(px)
    y0 = jnp.floor(py)
    wx1 = px - x0
    wx0 = 1.0 - wx1
    wy1 = py - y0
    wy0 = 1.0 - wy1
    vx0 = ((x0 >= 0.0) & (x0 < wv)).astype(jnp.float32)
    vx1 = ((x0 >= -1.0) & (x0 < wv - 1.0)).astype(jnp.float32)
    vy0 = ((y0 >= 0.0) & (y0 < hv)).astype(jnp.float32)
    vy1 = ((y0 >= -1.0) & (y0 < hv - 1.0)).astype(jnp.float32)
    wi = wv.astype(jnp.int32)
    xi0 = jnp.clip(x0, 0.0, wv - 1.0).astype(jnp.int32)
    xi1 = jnp.clip(x0 + 1.0, 0.0, wv - 1.0).astype(jnp.int32)
    yi0 = jnp.clip(y0, 0.0, hv - 1.0).astype(jnp.int32)
    yi1 = jnp.clip(y0 + 1.0, 0.0, hv - 1.0).astype(jnp.int32)
    bbase = b * NV
    r0 = base_ref[0] + yi0 * wi + bbase
    r1 = base_ref[0] + yi1 * wi + bbase
    hl8 = hlane_ref[0]
    iw_out[0, :, 0] = jnp.clip((r0 + xi0) * HEADS + hl8, 0, TOT_ROWS - 1)
    iw_out[0, :, 1] = jnp.clip((r0 + xi1) * HEADS + hl8, 0, TOT_ROWS - 1)
    iw_out[0, :, 2] = jnp.clip((r1 + xi0) * HEADS + hl8, 0, TOT_ROWS - 1)
    iw_out[0, :, 3] = jnp.clip((r1 + xi1) * HEADS + hl8, 0, TOT_ROWS - 1)
    bc = lambda x: lax.bitcast_convert_type(x, jnp.int32)
    iw_out[0, :, 4] = bc(aw * (wx0 * wy0) * (vx0 * vy0))
    iw_out[0, :, 5] = bc(aw * (wx1 * wy0) * (vx1 * vy0))
    iw_out[0, :, 6] = bc(aw * (wx0 * wy1) * (vx0 * vy1))
    iw_out[0, :, 7] = bc(aw * (wx1 * wy1) * (vx1 * vy1))


def _post_body(s_ref, w_ref, b_ref, q_ref, out_ref):
    out_ref[0] = (
        jnp.dot(s_ref[0], w_ref[...], preferred_element_type=jnp.float32)
        + b_ref[0] + q_ref[0])


def _sc_body(table, iwh, out, iw_v, rows_v, out_v, semg0, semg1, sem_o, sem_iw):
    wid = lax.axis_index("s") * 2 + lax.axis_index("c")
    semg = [semg0, semg1]
    srcrows = [qq * 8 + c4 for qq in range(2) for c4 in range(4)]

    def fire_iw(j, par):
        pltpu.async_copy(iwh.at[wid, j], iw_v.at[par], sem_iw)

    def wait_iw(par):
        pltpu.make_async_copy(iwh.at[wid, 0], iw_v.at[par], sem_iw).wait()

    def fire_gathers(par):
        for d in range(IDX_ROWS):
            pltpu.async_copy(table.at[iw_v.at[par, srcrows[d]]],
                             rows_v.at[par, pl.ds(d * 128, 128)], semg[par])

    def drain_gathers(par):
        for d in range(IDX_ROWS):
            pltpu.make_async_copy(
                table.at[iw_v.at[par, srcrows[d]]],
                rows_v.at[par, pl.ds(d * 128, 128)], semg[par]).wait()

    def wait_out(j, par):
        pltpu.make_async_copy(out_v.at[par], out.at[wid, j], sem_o).wait()

    fire_iw(0, 0)
    wait_iw(0)
    fire_gathers(0)
    fire_iw(1, 1)

    def outer(j2, carry):
        for par in range(2):
            j = j2 * 2 + par

            @pl.when(j < CH - 1)
            def _():
                wait_iw(1 - par)
                fire_gathers(1 - par)

            drain_gathers(par)

            def item(t, c2):
                qq = t >> 3
                h = t & 7
                a0 = jnp.zeros((16,), jnp.float32)
                b0 = jnp.zeros((16,), jnp.float32)
                a1 = jnp.zeros((16,), jnp.float32)
                b1 = jnp.zeros((16,), jnp.float32)
                for g in range(4):
                    col = h * 16
                    wv16 = plsc.bitcast(
                        iw_v[par, qq * 8 + 4 + g, pl.ds(col, 16)], jnp.float32)
                    rbase = (qq * 4 + g) * 128 + col
                    for k in range(16):
                        wk = wv16[k]
                        u = rows_v[par, rbase + k, :]
                        fe = plsc.bitcast(u << 16, jnp.float32)
                        fo = plsc.bitcast(u, jnp.float32)
                        if k & 1:
                            a1 = a1 + wk * fe
                            b1 = b1 + wk * fo
                        else:
                            a0 = a0 + wk * fe
                            b0 = b0 + wk * fo
                out_v[par, t, pl.ds(0, 16)] = a0 + a1
                out_v[par, t, pl.ds(16, 16)] = b0 + b1
                return c2

            lax.fori_loop(0, CI, item, 0)

            @pl.when(j + 2 < CH)
            def _():
                fire_iw(j + 2, par)

            @pl.when(j >= 2)
            def _():
                wait_out(j - 2, par)

            pltpu.async_copy(out_v.at[par], out.at[wid, j], sem_o)
        return carry

    lax.fori_loop(0, CH // 2, outer, 0)
    wait_out(CH - 2, 0)
    wait_out(CH - 1, 1)


def _fused_prep(value, query, reference_points, W_vp, b_vp, W_so, b_so,
                W_aw, b_aw):
    wall_t = jnp.concatenate([W_so[0::2], W_so[1::2], W_aw], axis=0).T
    ball = jnp.concatenate([b_so[0::2], b_so[1::2], b_aw]).reshape(1, 384)
    refx = jnp.tile(jnp.repeat(reference_points[..., 0], POINTS, axis=-1),
                    (1, 1, HEADS))
    refy = jnp.tile(jnp.repeat(reference_points[..., 1], POINTS, axis=-1),
                    (1, 1, HEADS))
    wl = jnp.asarray(_Wnp).reshape(1, 128)
    hl = jnp.asarray(_Hnp).reshape(1, 128)
    basev = jnp.asarray(_base).reshape(1, 128)
    hlane = jnp.asarray(_hlane).reshape(1, 128)
    bd = jnp.asarray(_BD)

    qspec = pl.BlockSpec((1, BQ, EMBED), lambda b, i: (b, i, 0))
    cspec128 = pl.BlockSpec((1, 128), lambda b, i: (0, 0))
    lspec = pl.BlockSpec((1, BQ, 128), lambda b, i: (b, i, 0))
    vtab, iw = pl.pallas_call(
        _fused_body,
        grid=(BS, NQB),
        in_specs=[
            qspec, qspec,
            pl.BlockSpec((EMBED, EMBED), lambda b, i: (0, 0)),
            pl.BlockSpec((1, EMBED), lambda b, i: (0, 0)),
            pl.BlockSpec((EMBED, 384), lambda b, i: (0, 0)),
            pl.BlockSpec((1, 384), lambda b, i: (0, 0)),
            lspec, lspec,
            cspec128, cspec128, cspec128, cspec128,
            pl.BlockSpec((128, 128), lambda b, i: (0, 0)),
        ],
        out_specs=[
            pl.BlockSpec((1, BQ, EMBED // 2), lambda b, i: (b, i, 0)),
            pl.BlockSpec((1, BQ, 8, 128), lambda b, i: (b, i, 0, 0)),
        ],
        out_shape=[
            jax.ShapeDtypeStruct((BS, NV, EMBED // 2), jnp.uint32),
            jax.ShapeDtypeStruct((BS, NQ, 8, 128), jnp.int32),
        ],
    )(value, query, W_vp.T, b_vp.reshape(1, EMBED), wall_t, ball,
      refx, refy, wl, hl, basev, hlane, bd)
    return vtab.reshape(TOT_ROWS, HD // 2), iw.reshape(NW, CH, 16, 128)


def _sc_gather(table, iw_sc):
    mesh = plsc.VectorSubcoreMesh(core_axis_name="c", subcore_axis_name="s")
    f = functools.partial(
        pl.kernel,
        mesh=mesh,
        compiler_params=pltpu.CompilerParams(use_tc_tiling_on_sc=False,
                                             needs_layout_passes=False),
        out_type=jax.ShapeDtypeStruct((NW, CH, CI, HD), jnp.float32),
        scratch_types=[
            pltpu.VMEM((2, 2 * IDX_ROWS, 128), jnp.int32),
            pltpu.VMEM((2, ROWS_PER_CHUNK, HD // 2), jnp.uint32),
            pltpu.VMEM((2, CI, HD), jnp.float32),
            pltpu.SemaphoreType.DMA,
            pltpu.SemaphoreType.DMA,
            pltpu.SemaphoreType.DMA,
            pltpu.SemaphoreType.DMA,
        ],
    )(_sc_body)
    return f(table, iw_sc)


def _post(sc_out, W_op, b_op, query):
    return pl.pallas_call(
        _post_body,
        grid=(BS, NQB),
        in_specs=[
            pl.BlockSpec((1, BQ, EMBED), lambda b, i: (b, i, 0)),
            pl.BlockSpec((EMBED, EMBED), lambda b, i: (0, 0)),
            pl.BlockSpec((1, EMBED), lambda b, i: (0, 0)),
            pl.BlockSpec((1, BQ, EMBED), lambda b, i: (b, i, 0)),
        ],
        out_specs=pl.BlockSpec((1, BQ, EMBED), lambda b, i: (b, i, 0)),
        out_shape=jax.ShapeDtypeStruct((BS, NQ, EMBED), jnp.float32),
    )(sc_out, W_op.T, b_op.reshape(1, EMBED), query)


def kernel(query, value, reference_points, spatial_shapes, level_start_index,
           W_so, b_so, W_aw, b_aw, W_vp, b_vp, W_op, b_op):
    table, iw_sc = _fused_prep(value, query, reference_points, W_vp, b_vp,
                               W_so, b_so, W_aw, b_aw)
    sc_out = _sc_gather(table, iw_sc)
    sc_out = sc_out.reshape(BS, NQ, EMBED)
    return _post(sc_out, W_op, b_op, query)


# speedup vs baseline: 6.1525x; 1.0962x over previous
import functools

import numpy as np
import jax
import jax.numpy as jnp
from jax import lax
from jax.experimental import pallas as pl
from jax.experimental.pallas import tpu as pltpu
from jax.experimental.pallas import tpu_sc as plsc

EMBED = 256
HEADS = 8
LEVELS = 4
POINTS = 4
HD = EMBED // HEADS
LP = LEVELS * POINTS
SPATIAL = [(64, 64), (32, 32), (16, 16), (8, 8)]
NV = sum(h * w for h, w in SPATIAL)
BS = 2
NQ = NV
TOT_ROWS = BS * NV * HEADS

BQ = 680
NQB = NQ // BQ

NW = 32
ITEMS = BS * NQ * HEADS
IW = ITEMS // NW
CI = 32
CH = IW // CI
ROWS_PER_CHUNK = CI * 4 * LP
IDX_ROWS = ROWS_PER_CHUNK // 128

_lvl = np.tile(np.repeat(np.arange(LEVELS), POINTS), HEADS)
_hh = np.repeat(np.arange(HEADS), LP)
_Wnp = np.array([w for (h, w) in SPATIAL], np.float32)[_lvl]
_Hnp = np.array([h for (h, w) in SPATIAL], np.float32)[_lvl]
_off = np.cumsum([0] + [h * w for h, w in SPATIAL])[:LEVELS]
_base = _off[_lvl].astype(np.int32)
_hlane = _hh.astype(np.int32)
_BD = np.kron(np.eye(HEADS, dtype=np.float32),
              np.ones((LP, LP), np.float32))


def _fused_body(x_ref, q_ref, wvp_ref, bvp_ref, wall_ref, ball_ref,
                refx_ref, refy_ref, wl_ref, hl_ref, base_ref, hlane_ref,
                bd_ref, v_out, iw_out):
    y = (jnp.dot(x_ref[0], wvp_ref[...], preferred_element_type=jnp.float32)
         + bvp_ref[0])
    u = lax.bitcast_convert_type(y, jnp.uint32)
    r = (u + jnp.uint32(0x7FFF) + ((u >> 16) & jnp.uint32(1))) >> 16
    packed = [r[:, h * HD:h * HD + 16] | (r[:, h * HD + 16:(h + 1) * HD] << 16)
              for h in range(HEADS)]
    v_out[0] = jnp.concatenate(packed, axis=1)

    b = pl.program_id(0)
    q = q_ref[0]
    t = jnp.dot(q, wall_ref[...], preferred_element_type=jnp.float32) + ball_ref[0]
    sox = t[:, 0:128]
    soy = t[:, 128:256]
    awr = t[:, 256:384]
    e = jnp.exp(awr)
    s = jnp.dot(e, bd_ref[...], preferred_element_type=jnp.float32)
    aw = e / s

    wv = wl_ref[0]
    hv = hl_ref[0]
    px = refx_ref[0] * wv + sox - 0.5
    py = refy_ref[0] * hv + soy - 0.5
    x0 = jnp.floor(px)
    y0 = jnp.floor(py)
    wx1 = px - x0
    wx0 = 1.0 - wx1
    wy1 = py - y0
    wy0 = 1.0 - wy1
    vx0 = ((x0 >= 0.0) & (x0 < wv)).astype(jnp.float32)
    vx1 = ((x0 >= -1.0) & (x0 < wv - 1.0)).astype(jnp.float32)
    vy0 = ((y0 >= 0.0) & (y0 < hv)).astype(jnp.float32)
    vy1 = ((y0 >= -1.0) & (y0 < hv - 1.0)).astype(jnp.float32)
    wi = wv.astype(jnp.int32)
    xi0 = jnp.clip(x0, 0.0, wv - 1.0).astype(jnp.int32)
    xi1 = jnp.clip(x0 + 1.0, 0.0, wv - 1.0).astype(jnp.int32)
    yi0 = jnp.clip(y0, 0.0, hv - 1.0).astype(jnp.int32)
    yi1 = jnp.clip(y0 + 1.0, 0.0, hv - 1.0).astype(jnp.int32)
    bbase = b * NV
    r0 = base_ref[0] + yi0 * wi + bbase
    r1 = base_ref[0] + yi1 * wi + bbase
    hl8 = hlane_ref[0]
    iw_out[0, :, 0] = jnp.clip((r0 + xi0) * HEADS + hl8, 0, TOT_ROWS - 1)
    iw_out[0, :, 1] = jnp.clip((r0 + xi1) * HEADS + hl8, 0, TOT_ROWS - 1)
    iw_out[0, :, 2] = jnp.clip((r1 + xi0) * HEADS + hl8, 0, TOT_ROWS - 1)
    iw_out[0, :, 3] = jnp.clip((r1 + xi1) * HEADS + hl8, 0, TOT_ROWS - 1)
    bc = lambda x: lax.bitcast_convert_type(x, jnp.int32)
    iw_out[0, :, 4] = bc(aw * (wx0 * wy0) * (vx0 * vy0))
    iw_out[0, :, 5] = bc(aw * (wx1 * wy0) * (vx1 * vy0))
    iw_out[0, :, 6] = bc(aw * (wx0 * wy1) * (vx0 * vy1))
    iw_out[0, :, 7] = bc(aw * (wx1 * wy1) * (vx1 * vy1))


def _post_body(s_ref, w_ref, b_ref, q_ref, out_ref):
    out_ref[0] = (
        jnp.dot(s_ref[0], w_ref[...], preferred_element_type=jnp.float32)
        + b_ref[0] + q_ref[0])


def _sc_body(table, iwh, out, iw_v, rows_v, out_v, semg0, semg1, sem_o, sem_iw):
    wid = lax.axis_index("s") * 2 + lax.axis_index("c")
    semg = [semg0, semg1]
    srcrows = [qq * 8 + c4 for qq in range(4) for c4 in range(4)]

    def fire_iw(j, par):
        pltpu.async_copy(iwh.at[wid, j], iw_v.at[par], sem_iw)

    def wait_iw(par):
        pltpu.make_async_copy(iwh.at[wid, 0], iw_v.at[par], sem_iw).wait()

    def fire_gathers(par):
        for d in range(IDX_ROWS):
            pltpu.async_copy(table.at[iw_v.at[par, srcrows[d]]],
                             rows_v.at[par, pl.ds(d * 128, 128)], semg[par])

    def drain_gathers(par):
        for d in range(IDX_ROWS):
            pltpu.make_async_copy(
                table.at[iw_v.at[par, srcrows[d]]],
                rows_v.at[par, pl.ds(d * 128, 128)], semg[par]).wait()

    def wait_out(j, par):
        pltpu.make_async_copy(out_v.at[par], out.at[wid, j], sem_o).wait()

    def chunk_step(j, par):
        @pl.when(j < CH - 1)
        def _():
            wait_iw(1 - par)
            fire_gathers(1 - par)

        drain_gathers(par)

        def item(t, c2):
            qq = t >> 3
            h = t & 7
            a0 = jnp.zeros((16,), jnp.float32)
            b0 = jnp.zeros((16,), jnp.float32)
            a1 = jnp.zeros((16,), jnp.float32)
            b1 = jnp.zeros((16,), jnp.float32)
            for g in range(4):
                col = h * 16
                wv16 = plsc.bitcast(
                    iw_v[par, qq * 8 + 4 + g, pl.ds(col, 16)], jnp.float32)
                rbase = (qq * 4 + g) * 128 + col
                for k in range(16):
                    wk = wv16[k]
                    u = rows_v[par, rbase + k, :]
                    fe = plsc.bitcast(u << 16, jnp.float32)
                    fo = plsc.bitcast(u, jnp.float32)
                    if k & 1:
                        a1 = a1 + wk * fe
                        b1 = b1 + wk * fo
                    else:
                        a0 = a0 + wk * fe
                        b0 = b0 + wk * fo
            out_v[par, t, pl.ds(0, 16)] = a0 + a1
            out_v[par, t, pl.ds(16, 16)] = b0 + b1
            return c2

        lax.fori_loop(0, CI, item, 0)

        @pl.when(j + 2 < CH)
        def _():
            fire_iw(j + 2, par)

        @pl.when(j >= 2)
        def _():
            wait_out(j - 2, par)

        pltpu.async_copy(out_v.at[par], out.at[wid, j], sem_o)

    fire_iw(0, 0)
    wait_iw(0)
    fire_gathers(0)
    fire_iw(1, 1)

    chunk_step(0, 0)

    def outer(j2, carry):
        j = 1 + j2 * 2
        chunk_step(j, 1)
        chunk_step(j + 1, 0)
        return carry

    lax.fori_loop(0, (CH - 1) // 2, outer, 0)
    wait_out(CH - 2, 1)
    wait_out(CH - 1, 0)


def _fused_prep(value, query, reference_points, W_vp, b_vp, W_so, b_so,
                W_aw, b_aw):
    wall_t = jnp.concatenate([W_so[0::2], W_so[1::2], W_aw], axis=0).T
    ball = jnp.concatenate([b_so[0::2], b_so[1::2], b_aw]).reshape(1, 384)
    refx = jnp.tile(jnp.repeat(reference_points[..., 0], POINTS, axis=-1),
                    (1, 1, HEADS))
    refy = jnp.tile(jnp.repeat(reference_points[..., 1], POINTS, axis=-1),
                    (1, 1, HEADS))
    wl = jnp.asarray(_Wnp).reshape(1, 128)
    hl = jnp.asarray(_Hnp).reshape(1, 128)
    basev = jnp.asarray(_base).reshape(1, 128)
    hlane = jnp.asarray(_hlane).reshape(1, 128)
    bd = jnp.asarray(_BD)

    qspec = pl.BlockSpec((1, BQ, EMBED), lambda b, i: (b, i, 0))
    cspec128 = pl.BlockSpec((1, 128), lambda b, i: (0, 0))
    lspec = pl.BlockSpec((1, BQ, 128), lambda b, i: (b, i, 0))
    vtab, iw = pl.pallas_call(
        _fused_body,
        grid=(BS, NQB),
        in_specs=[
            qspec, qspec,
            pl.BlockSpec((EMBED, EMBED), lambda b, i: (0, 0)),
            pl.BlockSpec((1, EMBED), lambda b, i: (0, 0)),
            pl.BlockSpec((EMBED, 384), lambda b, i: (0, 0)),
            pl.BlockSpec((1, 384), lambda b, i: (0, 0)),
            lspec, lspec,
            cspec128, cspec128, cspec128, cspec128,
            pl.BlockSpec((128, 128), lambda b, i: (0, 0)),
        ],
        out_specs=[
            pl.BlockSpec((1, BQ, EMBED // 2), lambda b, i: (b, i, 0)),
            pl.BlockSpec((1, BQ, 8, 128), lambda b, i: (b, i, 0, 0)),
        ],
        out_shape=[
            jax.ShapeDtypeStruct((BS, NV, EMBED // 2), jnp.uint32),
            jax.ShapeDtypeStruct((BS, NQ, 8, 128), jnp.int32),
        ],
    )(value, query, W_vp.T, b_vp.reshape(1, EMBED), wall_t, ball,
      refx, refy, wl, hl, basev, hlane, bd)
    return vtab.reshape(TOT_ROWS, HD // 2), iw.reshape(NW, CH, 2 * IDX_ROWS, 128)


def _sc_gather(table, iw_sc):
    mesh = plsc.VectorSubcoreMesh(core_axis_name="c", subcore_axis_name="s")
    f = functools.partial(
        pl.kernel,
        mesh=mesh,
        compiler_params=pltpu.CompilerParams(use_tc_tiling_on_sc=False,
                                             needs_layout_passes=False),
        out_type=jax.ShapeDtypeStruct((NW, CH, CI, HD), jnp.float32),
        scratch_types=[
            pltpu.VMEM((2, 2 * IDX_ROWS, 128), jnp.int32),
            pltpu.VMEM((2, ROWS_PER_CHUNK, HD // 2), jnp.uint32),
            pltpu.VMEM((2, CI, HD), jnp.float32),
            pltpu.SemaphoreType.DMA,
            pltpu.SemaphoreType.DMA,
            pltpu.SemaphoreType.DMA,
            pltpu.SemaphoreType.DMA,
        ],
    )(_sc_body)
    return f(table, iw_sc)


def _post(sc_out, W_op, b_op, query):
    return pl.pallas_call(
        _post_body,
        grid=(BS, NQB),
        in_specs=[
            pl.BlockSpec((1, BQ, EMBED), lambda b, i: (b, i, 0)),
            pl.BlockSpec((EMBED, EMBED), lambda b, i: (0, 0)),
            pl.BlockSpec((1, EMBED), lambda b, i: (0, 0)),
            pl.BlockSpec((1, BQ, EMBED), lambda b, i: (b, i, 0)),
        ],
        out_specs=pl.BlockSpec((1, BQ, EMBED), lambda b, i: (b, i, 0)),
        out_shape=jax.ShapeDtypeStruct((BS, NQ, EMBED), jnp.float32),
    )(sc_out, W_op.T, b_op.reshape(1, EMBED), query)


def kernel(query, value, reference_points, spatial_shapes, level_start_index,
           W_so, b_so, W_aw, b_aw, W_vp, b_vp, W_op, b_op):
    table, iw_sc = _fused_prep(value, query, reference_points, W_vp, b_vp,
                               W_so, b_so, W_aw, b_aw)
    sc_out = _sc_gather(table, iw_sc)
    sc_out = sc_out.reshape(BS, NQ, EMBED)
    return _post(sc_out, W_op, b_op, query)


# speedup vs baseline: 6.3234x; 1.0278x over previous
import functools

import numpy as np
import jax
import jax.numpy as jnp
from jax import lax
from jax.experimental import pallas as pl
from jax.experimental.pallas import tpu as pltpu
from jax.experimental.pallas import tpu_sc as plsc

EMBED = 256
HEADS = 8
LEVELS = 4
POINTS = 4
HD = EMBED // HEADS
LP = LEVELS * POINTS
SPATIAL = [(64, 64), (32, 32), (16, 16), (8, 8)]
NV = sum(h * w for h, w in SPATIAL)
BS = 2
NQ = NV
TOT_ROWS = BS * NV * HEADS

BQ = 680
NQB = NQ // BQ

NW = 32
ITEMS = BS * NQ * HEADS
IW = ITEMS // NW
CI = 40
CH = IW // CI
ROWS_PER_CHUNK = CI * 4 * LP
IDX_ROWS = ROWS_PER_CHUNK // 128

_lvl = np.tile(np.repeat(np.arange(LEVELS), POINTS), HEADS)
_hh = np.repeat(np.arange(HEADS), LP)
_Wnp = np.array([w for (h, w) in SPATIAL], np.float32)[_lvl]
_Hnp = np.array([h for (h, w) in SPATIAL], np.float32)[_lvl]
_off = np.cumsum([0] + [h * w for h, w in SPATIAL])[:LEVELS]
_base = _off[_lvl].astype(np.int32)
_hlane = _hh.astype(np.int32)
_BD = np.kron(np.eye(HEADS, dtype=np.float32),
              np.ones((LP, LP), np.float32))


def _fused_body(x_ref, q_ref, wvp_ref, bvp_ref, wall_ref, ball_ref,
                refx_ref, refy_ref, wl_ref, hl_ref, base_ref, hlane_ref,
                bd_ref, v_out, iw_out):
    y = (jnp.dot(x_ref[0], wvp_ref[...], preferred_element_type=jnp.float32)
         + bvp_ref[0])
    u = lax.bitcast_convert_type(y, jnp.uint32)
    r = (u + jnp.uint32(0x7FFF) + ((u >> 16) & jnp.uint32(1))) >> 16
    packed = [r[:, h * HD:h * HD + 16] | (r[:, h * HD + 16:(h + 1) * HD] << 16)
              for h in range(HEADS)]
    v_out[0] = jnp.concatenate(packed, axis=1)

    b = pl.program_id(0)
    q = q_ref[0]
    t = jnp.dot(q, wall_ref[...], preferred_element_type=jnp.float32) + ball_ref[0]
    sox = t[:, 0:128]
    soy = t[:, 128:256]
    awr = t[:, 256:384]
    e = jnp.exp(awr)
    s = jnp.dot(e, bd_ref[...], preferred_element_type=jnp.float32)
    aw = e / s

    wv = wl_ref[0]
    hv = hl_ref[0]
    px = refx_ref[0] * wv + sox - 0.5
    py = refy_ref[0] * hv + soy - 0.5
    x0 = jnp.floor(px)
    y0 = jnp.floor(py)
    wx1 = px - x0
    wx0 = 1.0 - wx1
    wy1 = py - y0
    wy0 = 1.0 - wy1
    vx0 = ((x0 >= 0.0) & (x0 < wv)).astype(jnp.float32)
    vx1 = ((x0 >= -1.0) & (x0 < wv - 1.0)).astype(jnp.float32)
    vy0 = ((y0 >= 0.0) & (y0 < hv)).astype(jnp.float32)
    vy1 = ((y0 >= -1.0) & (y0 < hv - 1.0)).astype(jnp.float32)
    wi = wv.astype(jnp.int32)
    xi0 = jnp.clip(x0, 0.0, wv - 1.0).astype(jnp.int32)
    xi1 = jnp.clip(x0 + 1.0, 0.0, wv - 1.0).astype(jnp.int32)
    yi0 = jnp.clip(y0, 0.0, hv - 1.0).astype(jnp.int32)
    yi1 = jnp.clip(y0 + 1.0, 0.0, hv - 1.0).astype(jnp.int32)
    bbase = b * NV
    r0 = base_ref[0] + yi0 * wi + bbase
    r1 = base_ref[0] + yi1 * wi + bbase
    hl8 = hlane_ref[0]
    iw_out[0, :, 0] = jnp.clip((r0 + xi0) * HEADS + hl8, 0, TOT_ROWS - 1)
    iw_out[0, :, 1] = jnp.clip((r0 + xi1) * HEADS + hl8, 0, TOT_ROWS - 1)
    iw_out[0, :, 2] = jnp.clip((r1 + xi0) * HEADS + hl8, 0, TOT_ROWS - 1)
    iw_out[0, :, 3] = jnp.clip((r1 + xi1) * HEADS + hl8, 0, TOT_ROWS - 1)
    bc = lambda x: lax.bitcast_convert_type(x, jnp.int32)
    iw_out[0, :, 4] = bc(aw * (wx0 * wy0) * (vx0 * vy0))
    iw_out[0, :, 5] = bc(aw * (wx1 * wy0) * (vx1 * vy0))
    iw_out[0, :, 6] = bc(aw * (wx0 * wy1) * (vx0 * vy1))
    iw_out[0, :, 7] = bc(aw * (wx1 * wy1) * (vx1 * vy1))


def _post_body(s_ref, w_ref, b_ref, q_ref, out_ref):
    out_ref[0] = (
        jnp.dot(s_ref[0], w_ref[...], preferred_element_type=jnp.float32)
        + b_ref[0] + q_ref[0])


def _sc_body(table, iwh, out, iw_v, rows_v, out_v, semg0, semg1, sem_o, sem_iw):
    wid = lax.axis_index("s") * 2 + lax.axis_index("c")
    semg = [semg0, semg1]
    srcrows = [qq * 8 + c4 for qq in range(CI // 8) for c4 in range(4)]

    def fire_iw(j, par):
        pltpu.async_copy(iwh.at[wid, j], iw_v.at[par], sem_iw)

    def wait_iw(par):
        pltpu.make_async_copy(iwh.at[wid, 0], iw_v.at[par], sem_iw).wait()

    def fire_gathers(par):
        for d in range(IDX_ROWS):
            pltpu.async_copy(table.at[iw_v.at[par, srcrows[d]]],
                             rows_v.at[par, pl.ds(d * 128, 128)], semg[par])

    def drain_gathers(par):
        for d in range(IDX_ROWS):
            pltpu.make_async_copy(
                table.at[iw_v.at[par, srcrows[d]]],
                rows_v.at[par, pl.ds(d * 128, 128)], semg[par]).wait()

    def wait_out(j, par):
        pltpu.make_async_copy(out_v.at[par], out.at[wid, j], sem_o).wait()

    def chunk_step(j, par):
        @pl.when(j < CH - 1)
        def _():
            wait_iw(1 - par)
            fire_gathers(1 - par)

        drain_gathers(par)

        def item(t, c2):
            qq = t >> 3
            h = t & 7
            a0 = jnp.zeros((16,), jnp.float32)
            b0 = jnp.zeros((16,), jnp.float32)
            a1 = jnp.zeros((16,), jnp.float32)
            b1 = jnp.zeros((16,), jnp.float32)
            for g in range(4):
                col = h * 16
                wv16 = plsc.bitcast(
                    iw_v[par, qq * 8 + 4 + g, pl.ds(col, 16)], jnp.float32)
                rbase = (qq * 4 + g) * 128 + col
                for k in range(16):
                    wk = wv16[k]
                    u = rows_v[par, rbase + k, :]
                    fe = plsc.bitcast(u << 16, jnp.float32)
                    fo = plsc.bitcast(u, jnp.float32)
                    if k & 1:
                        a1 = a1 + wk * fe
                        b1 = b1 + wk * fo
                    else:
                        a0 = a0 + wk * fe
                        b0 = b0 + wk * fo
            out_v[par, t, pl.ds(0, 16)] = a0 + a1
            out_v[par, t, pl.ds(16, 16)] = b0 + b1
            return c2

        lax.fori_loop(0, CI, item, 0)

        @pl.when(j + 2 < CH)
        def _():
            fire_iw(j + 2, par)

        @pl.when(j >= 2)
        def _():
            wait_out(j - 2, par)

        pltpu.async_copy(out_v.at[par], out.at[wid, j], sem_o)

    fire_iw(0, 0)
    wait_iw(0)
    fire_gathers(0)
    fire_iw(1, 1)

    if CH % 2:
        chunk_step(0, 0)

        def outer(j2, carry):
            j = 1 + j2 * 2
            chunk_step(j, 1)
            chunk_step(j + 1, 0)
            return carry

        lax.fori_loop(0, (CH - 1) // 2, outer, 0)
        wait_out(CH - 2, 1)
        wait_out(CH - 1, 0)
    else:
        def outer(j2, carry):
            j = j2 * 2
            chunk_step(j, 0)
            chunk_step(j + 1, 1)
            return carry

        lax.fori_loop(0, CH // 2, outer, 0)
        wait_out(CH - 2, 0)
        wait_out(CH - 1, 1)


def _fused_prep(value, query, reference_points, W_vp, b_vp, W_so, b_so,
                W_aw, b_aw):
    wall_t = jnp.concatenate([W_so[0::2], W_so[1::2], W_aw], axis=0).T
    ball = jnp.concatenate([b_so[0::2], b_so[1::2], b_aw]).reshape(1, 384)
    refx = jnp.tile(jnp.repeat(reference_points[..., 0], POINTS, axis=-1),
                    (1, 1, HEADS))
    refy = jnp.tile(jnp.repeat(reference_points[..., 1], POINTS, axis=-1),
                    (1, 1, HEADS))
    wl = jnp.asarray(_Wnp).reshape(1, 128)
    hl = jnp.asarray(_Hnp).reshape(1, 128)
    basev = jnp.asarray(_base).reshape(1, 128)
    hlane = jnp.asarray(_hlane).reshape(1, 128)
    bd = jnp.asarray(_BD)

    qspec = pl.BlockSpec((1, BQ, EMBED), lambda b, i: (b, i, 0))
    cspec128 = pl.BlockSpec((1, 128), lambda b, i: (0, 0))
    lspec = pl.BlockSpec((1, BQ, 128), lambda b, i: (b, i, 0))
    vtab, iw = pl.pallas_call(
        _fused_body,
        grid=(BS, NQB),
        in_specs=[
            qspec, qspec,
            pl.BlockSpec((EMBED, EMBED), lambda b, i: (0, 0)),
            pl.BlockSpec((1, EMBED), lambda b, i: (0, 0)),
            pl.BlockSpec((EMBED, 384), lambda b, i: (0, 0)),
            pl.BlockSpec((1, 384), lambda b, i: (0, 0)),
            lspec, lspec,
            cspec128, cspec128, cspec128, cspec128,
            pl.BlockSpec((128, 128), lambda b, i: (0, 0)),
        ],
        out_specs=[
            pl.BlockSpec((1, BQ, EMBED // 2), lambda b, i: (b, i, 0)),
            pl.BlockSpec((1, BQ, 8, 128), lambda b, i: (b, i, 0, 0)),
        ],
        out_shape=[
            jax.ShapeDtypeStruct((BS, NV, EMBED // 2), jnp.uint32),
            jax.ShapeDtypeStruct((BS, NQ, 8, 128), jnp.int32),
        ],
    )(value, query, W_vp.T, b_vp.reshape(1, EMBED), wall_t, ball,
      refx, refy, wl, hl, basev, hlane, bd)
    return vtab.reshape(TOT_ROWS, HD // 2), iw.reshape(NW, CH, 2 * IDX_ROWS, 128)


def _sc_gather(table, iw_sc):
    mesh = plsc.VectorSubcoreMesh(core_axis_name="c", subcore_axis_name="s")
    f = functools.partial(
        pl.kernel,
        mesh=mesh,
        compiler_params=pltpu.CompilerParams(use_tc_tiling_on_sc=False,
                                             needs_layout_passes=False),
        out_type=jax.ShapeDtypeStruct((NW, CH, CI, HD), jnp.float32),
        scratch_types=[
            pltpu.VMEM((2, 2 * IDX_ROWS, 128), jnp.int32),
            pltpu.VMEM((2, ROWS_PER_CHUNK, HD // 2), jnp.uint32),
            pltpu.VMEM((2, CI, HD), jnp.float32),
            pltpu.SemaphoreType.DMA,
            pltpu.SemaphoreType.DMA,
            pltpu.SemaphoreType.DMA,
            pltpu.SemaphoreType.DMA,
        ],
    )(_sc_body)
    return f(table, iw_sc)


def _post(sc_out, W_op, b_op, query):
    return pl.pallas_call(
        _post_body,
        grid=(BS, NQB),
        in_specs=[
            pl.BlockSpec((1, BQ, EMBED), lambda b, i: (b, i, 0)),
            pl.BlockSpec((EMBED, EMBED), lambda b, i: (0, 0)),
            pl.BlockSpec((1, EMBED), lambda b, i: (0, 0)),
            pl.BlockSpec((1, BQ, EMBED), lambda b, i: (b, i, 0)),
        ],
        out_specs=pl.BlockSpec((1, BQ, EMBED), lambda b, i: (b, i, 0)),
        out_shape=jax.ShapeDtypeStruct((BS, NQ, EMBED), jnp.float32),
    )(sc_out, W_op.T, b_op.reshape(1, EMBED), query)


def kernel(query, value, reference_points, spatial_shapes, level_start_index,
           W_so, b_so, W_aw, b_aw, W_vp, b_vp, W_op, b_op):
    table, iw_sc = _fused_prep(value, query, reference_points, W_vp, b_vp,
                               W_so, b_so, W_aw, b_aw)
    sc_out = _sc_gather(table, iw_sc)
    sc_out = sc_out.reshape(BS, NQ, EMBED)
    return _post(sc_out, W_op, b_op, query)
